# Initial kernel scaffold; baseline (speedup 1.0000x reference)
import jax
import jax.numpy as jnp
from jax.experimental import pallas as pl


def kernel(x, token_type_ids, word_table, pos_table, type_table, gamma, beta):
    raise NotImplementedError("write your pallas kernel here")


# speedup vs baseline: 3.0410x; 3.0410x over previous
import functools

import jax
import jax.numpy as jnp
from jax import lax
from jax.experimental import pallas as pl
from jax.experimental.pallas import tpu as pltpu
from jax.experimental.pallas import tpu_sc as plsc

B = 1024
S = 512
D = 128
MAXPOS = 512
EPS = 1e-12
L = 16
NC = 2
NS = 16
NW = NC * NS
T = 128
TOK_PER_TILE = (B // NW) * S
NCHUNK = TOK_PER_TILE // T
SCHUNK = S // T


_GATHER_DNUMS = lax.GatherDimensionNumbers(
    offset_dims=(), collapsed_slice_dims=(0,), start_index_map=(0,))


def _shuffle(v, idx):
    return lax.gather(v, idx[:, None], _GATHER_DNUMS, slice_sizes=(1,),
                      mode=lax.GatherScatterMode.PROMISE_IN_BOUNDS)


def _allsum(v):
    for k in (8, 4, 2, 1):
        idx = jnp.bitwise_xor(lax.iota(jnp.int32, 16), k)
        v = v + _shuffle(v, idx)
    return v


def _rsqrt_nr(v):
    i = lax.bitcast_convert_type(v, jnp.int32)
    i = 0x5F3759DF - lax.shift_right_logical(i, 1)
    y = lax.bitcast_convert_type(i, jnp.float32)
    half = v * 0.5
    for _ in range(3):
        y = y * (1.5 - half * y * y)
    return y


_mesh = plsc.VectorSubcoreMesh(core_axis_name="c", subcore_axis_name="s")


@functools.partial(
    pl.kernel,
    out_type=jax.ShapeDtypeStruct((B * S, D), jnp.float32),
    mesh=_mesh,
    scratch_types=[
        pltpu.VMEM((MAXPOS, D), jnp.float32),
        pltpu.VMEM((T, D), jnp.float32),
        pltpu.VMEM((T,), jnp.int32),
        pltpu.VMEM((D,), jnp.float32),
        pltpu.VMEM((D,), jnp.float32),
        pltpu.VMEM((D,), jnp.float32),
        pltpu.VMEM((T,), jnp.float32),
        pltpu.SemaphoreType.DMA,
    ],
)
def _emb_kernel(ids_hbm, tf_hbm, word_hbm, p0_hbm, dvec_hbm, gamma_hbm,
                beta_hbm, out_hbm, p0_v, rows_v, idx_v, dvec_v, gamma_v,
                beta_v, tf_v, sem):
    wid = lax.axis_index("s") * NC + lax.axis_index("c")
    tile_base = wid * TOK_PER_TILE

    pltpu.sync_copy(p0_hbm, p0_v)
    pltpu.sync_copy(dvec_hbm, dvec_v)
    pltpu.sync_copy(gamma_hbm, gamma_v)
    pltpu.sync_copy(beta_hbm, beta_v)
    g8 = [gamma_v[pl.ds(L * j, L)] for j in range(8)]
    b8 = [beta_v[pl.ds(L * j, L)] for j in range(8)]
    d8 = [dvec_v[pl.ds(L * j, L)] for j in range(8)]

    def chunk_body(g, carry):
        base = tile_base + g * T
        s_base = (g % SCHUNK) * T
        pltpu.sync_copy(ids_hbm.at[pl.ds(base, T)], idx_v)
        pltpu.sync_copy(tf_hbm.at[pl.ds(base, T)], tf_v)
        pltpu.async_copy(word_hbm.at[idx_v], rows_v, sem).wait()

        def grp_body(t, c2):
            i0 = t * L
            tfblk = tf_v[pl.ds(i0, L)]
            for lane in range(L):
                i = i0 + lane
                s = s_base + i
                tfv = _shuffle(tfblk, jnp.full((L,), lane, dtype=jnp.int32))
                h = []
                for j in range(8):
                    w = rows_v[i, pl.ds(L * j, L)]
                    p = p0_v[s, pl.ds(L * j, L)]
                    h.append(w + p + tfv * d8[j])
                tot = (((h[0] + h[1]) + (h[2] + h[3]))
                       + ((h[4] + h[5]) + (h[6] + h[7])))
                sq = [hj * hj for hj in h]
                qot = (((sq[0] + sq[1]) + (sq[2] + sq[3]))
                       + ((sq[4] + sq[5]) + (sq[6] + sq[7])))
                mv = _allsum(tot) * (1.0 / D)
                q = _allsum(qot) * (1.0 / D)
                var = q - mv * mv
                rstd = _rsqrt_nr(var + EPS)
                for j in range(8):
                    rows_v[i, pl.ds(L * j, L)] = (h[j] - mv) * rstd * g8[j] + b8[j]
            return c2

        lax.fori_loop(0, T // L, grp_body, 0)
        pltpu.sync_copy(rows_v, out_hbm.at[pl.ds(base, T)])
        return carry

    lax.fori_loop(0, NCHUNK, chunk_body, 0)


def kernel(x, token_type_ids, word_table, pos_table, type_table, gamma, beta):
    ids = x.reshape(-1).astype(jnp.int32)
    tf = token_type_ids.reshape(-1).astype(jnp.float32)
    p0 = pos_table + type_table[0]
    dvec = type_table[1] - type_table[0]
    out = _emb_kernel(ids, tf, word_table, p0, dvec, gamma, beta)
    return out.reshape(B, S, D)


# speedup vs baseline: 3.6748x; 1.2084x over previous
import functools

import jax
import jax.numpy as jnp
from jax import lax
from jax.experimental import pallas as pl
from jax.experimental.pallas import tpu as pltpu
from jax.experimental.pallas import tpu_sc as plsc

B = 1024
S = 512
D = 128
MAXPOS = 512
EPS = 1e-12
L = 16
NC = 2
NS = 16
NW = NC * NS
T = 128
TOK_PER_TILE = (B // NW) * S
NCHUNK = TOK_PER_TILE // T
SCHUNK = S // T

_GATHER_DNUMS = lax.GatherDimensionNumbers(
    offset_dims=(), collapsed_slice_dims=(0,), start_index_map=(0,))


def _shuffle(v, idx):
    return lax.gather(v, idx[:, None], _GATHER_DNUMS, slice_sizes=(1,),
                      mode=lax.GatherScatterMode.PROMISE_IN_BOUNDS)


def _allsum(v):
    for k in (8, 4, 2, 1):
        idx = jnp.bitwise_xor(lax.iota(jnp.int32, 16), k)
        v = v + _shuffle(v, idx)
    return v


def _rsqrt_nr(v):
    i = lax.bitcast_convert_type(v, jnp.int32)
    i = 0x5F3759DF - lax.shift_right_logical(i, 1)
    y = lax.bitcast_convert_type(i, jnp.float32)
    half = v * 0.5
    for _ in range(2):
        y = y * (1.5 - half * y * y)
    return y


_mesh = plsc.VectorSubcoreMesh(core_axis_name="c", subcore_axis_name="s")


@functools.partial(
    pl.kernel,
    out_type=jax.ShapeDtypeStruct((B * S, D), jnp.float32),
    mesh=_mesh,
    scratch_types=[
        pltpu.VMEM((MAXPOS, D), jnp.float32),
        pltpu.VMEM((T, D), jnp.float32),
        pltpu.VMEM((T, D), jnp.float32),
        pltpu.VMEM((T,), jnp.int32),
        pltpu.VMEM((T,), jnp.int32),
        pltpu.VMEM((T,), jnp.float32),
        pltpu.VMEM((T,), jnp.float32),
        pltpu.VMEM((D,), jnp.float32),
        pltpu.VMEM((D,), jnp.float32),
        pltpu.VMEM((D,), jnp.float32),
        pltpu.SemaphoreType.DMA,
        pltpu.SemaphoreType.DMA,
    ],
)
def _emb_kernel(ids_hbm, tf_hbm, word_hbm, p0_hbm, dvec_hbm, gamma_hbm,
                beta_hbm, out_hbm, p0_v, rows0, rows1, idx0, idx1, tf0, tf1,
                dvec_v, gamma_v, beta_v, sem0, sem1):
    wid = lax.axis_index("s") * NC + lax.axis_index("c")
    tile_base = wid * TOK_PER_TILE

    pltpu.sync_copy(p0_hbm, p0_v)
    pltpu.sync_copy(dvec_hbm, dvec_v)
    pltpu.sync_copy(gamma_hbm, gamma_v)
    pltpu.sync_copy(beta_hbm, beta_v)
    g8 = [gamma_v[pl.ds(L * j, L)] for j in range(8)]
    b8 = [beta_v[pl.ds(L * j, L)] for j in range(8)]
    d8 = [dvec_v[pl.ds(L * j, L)] for j in range(8)]

    def issue(c, idx_v, tf_v, rows_v, sem):
        base = tile_base + c * T
        pltpu.sync_copy(ids_hbm.at[pl.ds(base, T)], idx_v)
        pltpu.sync_copy(tf_hbm.at[pl.ds(base, T)], tf_v)
        pltpu.async_copy(word_hbm.at[idx_v], rows_v, sem)

    def compute(c, idx_v, tf_v, rows_v, sem):
        pltpu.make_async_copy(word_hbm.at[idx_v], rows_v, sem).wait()
        s_base = (c % SCHUNK) * T

        def grp_body(t, c2):
            i0 = t * L
            tfblk = tf_v[pl.ds(i0, L)]
            for lane in range(L):
                i = i0 + lane
                s = s_base + i
                tfv = _shuffle(tfblk, jnp.full((L,), lane, dtype=jnp.int32))
                h = []
                for j in range(8):
                    w = rows_v[i, pl.ds(L * j, L)]
                    p = p0_v[s, pl.ds(L * j, L)]
                    h.append(w + p + tfv * d8[j])
                tot = (((h[0] + h[1]) + (h[2] + h[3]))
                       + ((h[4] + h[5]) + (h[6] + h[7])))
                sq = [hj * hj for hj in h]
                qot = (((sq[0] + sq[1]) + (sq[2] + sq[3]))
                       + ((sq[4] + sq[5]) + (sq[6] + sq[7])))
                mv = _allsum(tot) * (1.0 / D)
                q = _allsum(qot) * (1.0 / D)
                var = q - mv * mv
                rstd = _rsqrt_nr(var + EPS)
                for j in range(8):
                    rows_v[i, pl.ds(L * j, L)] = (h[j] - mv) * rstd * g8[j] + b8[j]
            return c2

        lax.fori_loop(0, T // L, grp_body, 0)
        pltpu.sync_copy(rows_v, out_hbm.at[pl.ds(tile_base + c * T, T)])

    issue(0, idx0, tf0, rows0, sem0)

    def body(g2, carry):
        c0 = 2 * g2
        issue(c0 + 1, idx1, tf1, rows1, sem1)
        compute(c0, idx0, tf0, rows0, sem0)

        @pl.when(c0 + 2 < NCHUNK)
        def _():
            issue(c0 + 2, idx0, tf0, rows0, sem0)

        compute(c0 + 1, idx1, tf1, rows1, sem1)
        return carry

    lax.fori_loop(0, NCHUNK // 2, body, 0)


def kernel(x, token_type_ids, word_table, pos_table, type_table, gamma, beta):
    ids = x.reshape(-1).astype(jnp.int32)
    tf = token_type_ids.reshape(-1).astype(jnp.float32)
    p0 = pos_table + type_table[0]
    dvec = type_table[1] - type_table[0]
    out = _emb_kernel(ids, tf, word_table, p0, dvec, gamma, beta)
    return out.reshape(B, S, D)


# speedup vs baseline: 3.6821x; 1.0020x over previous
import functools

import jax
import jax.numpy as jnp
from jax import lax
from jax.experimental import pallas as pl
from jax.experimental.pallas import tpu as pltpu
from jax.experimental.pallas import tpu_sc as plsc

B = 1024
S = 512
D = 128
MAXPOS = 512
EPS = 1e-12
L = 16
NC = 2
NS = 16
NW = NC * NS
T = 128
TOK_PER_TILE = (B // NW) * S
NCHUNK = TOK_PER_TILE // T
SCHUNK = S // T

_GATHER_DNUMS = lax.GatherDimensionNumbers(
    offset_dims=(), collapsed_slice_dims=(0,), start_index_map=(0,))


def _shuffle(v, idx):
    return lax.gather(v, idx[:, None], _GATHER_DNUMS, slice_sizes=(1,),
                      mode=lax.GatherScatterMode.PROMISE_IN_BOUNDS)


def _allsum(v):
    for k in (8, 4, 2, 1):
        idx = jnp.bitwise_xor(lax.iota(jnp.int32, 16), k)
        v = v + _shuffle(v, idx)
    return v


def _rsqrt_nr(v):
    i = lax.bitcast_convert_type(v, jnp.int32)
    i = 0x5F3759DF - lax.shift_right_logical(i, 1)
    y = lax.bitcast_convert_type(i, jnp.float32)
    half = v * 0.5
    for _ in range(2):
        y = y * (1.5 - half * y * y)
    return y


_mesh = plsc.VectorSubcoreMesh(core_axis_name="c", subcore_axis_name="s")


@functools.partial(
    pl.kernel,
    out_type=jax.ShapeDtypeStruct((B * S, D), jnp.float32),
    mesh=_mesh,
    scratch_types=[
        pltpu.VMEM((MAXPOS, D), jnp.float32),
        pltpu.VMEM((T, D), jnp.float32),
        pltpu.VMEM((T, D), jnp.float32),
        pltpu.VMEM((T,), jnp.int32),
        pltpu.VMEM((T,), jnp.int32),
        pltpu.VMEM((T,), jnp.float32),
        pltpu.VMEM((T,), jnp.float32),
        pltpu.VMEM((D,), jnp.float32),
        pltpu.VMEM((D,), jnp.float32),
        pltpu.VMEM((D,), jnp.float32),
        pltpu.SemaphoreType.DMA,
        pltpu.SemaphoreType.DMA,
        pltpu.SemaphoreType.DMA,
        pltpu.SemaphoreType.DMA,
    ],
)
def _emb_kernel(ids_hbm, tf_hbm, word_hbm, p0_hbm, dvec_hbm, gamma_hbm,
                beta_hbm, out_hbm, p0_v, rows0, rows1, idx0, idx1, tf0, tf1,
                dvec_v, gamma_v, beta_v, sem0, sem1, osem0, osem1):
    wid = lax.axis_index("s") * NC + lax.axis_index("c")
    tile_base = wid * TOK_PER_TILE

    pltpu.sync_copy(p0_hbm, p0_v)
    pltpu.sync_copy(dvec_hbm, dvec_v)
    pltpu.sync_copy(gamma_hbm, gamma_v)
    pltpu.sync_copy(beta_hbm, beta_v)
    g8 = [gamma_v[pl.ds(L * j, L)] for j in range(8)]
    b8 = [beta_v[pl.ds(L * j, L)] for j in range(8)]
    d8 = [dvec_v[pl.ds(L * j, L)] for j in range(8)]

    def issue(c, idx_v, tf_v, rows_v, sem, osem):
        @pl.when(c > 1)
        def _():
            pltpu.make_async_copy(
                rows_v, out_hbm.at[pl.ds(tile_base, T)], osem).wait()
        base = tile_base + c * T
        pltpu.sync_copy(ids_hbm.at[pl.ds(base, T)], idx_v)
        pltpu.sync_copy(tf_hbm.at[pl.ds(base, T)], tf_v)
        pltpu.async_copy(word_hbm.at[idx_v], rows_v, sem)

    def compute(c, idx_v, tf_v, rows_v, sem, osem):
        pltpu.make_async_copy(word_hbm.at[idx_v], rows_v, sem).wait()
        s_base = (c % SCHUNK) * T

        def grp_body(t, c2):
            i0 = t * L
            tfblk = tf_v[pl.ds(i0, L)]
            for lane in range(L):
                i = i0 + lane
                s = s_base + i
                tfv = _shuffle(tfblk, jnp.full((L,), lane, dtype=jnp.int32))
                h = []
                for j in range(8):
                    w = rows_v[i, pl.ds(L * j, L)]
                    p = p0_v[s, pl.ds(L * j, L)]
                    h.append(w + p + tfv * d8[j])
                tot = (((h[0] + h[1]) + (h[2] + h[3]))
                       + ((h[4] + h[5]) + (h[6] + h[7])))
                sq = [hj * hj for hj in h]
                qot = (((sq[0] + sq[1]) + (sq[2] + sq[3]))
                       + ((sq[4] + sq[5]) + (sq[6] + sq[7])))
                mv = _allsum(tot) * (1.0 / D)
                q = _allsum(qot) * (1.0 / D)
                var = q - mv * mv
                rstd = _rsqrt_nr(var + EPS)
                for j in range(8):
                    rows_v[i, pl.ds(L * j, L)] = (h[j] - mv) * rstd * g8[j] + b8[j]
            return c2

        lax.fori_loop(0, T // L, grp_body, 0)
        pltpu.async_copy(rows_v, out_hbm.at[pl.ds(tile_base + c * T, T)], osem)

    issue(0, idx0, tf0, rows0, sem0, osem0)

    def body(g2, carry):
        c0 = 2 * g2
        issue(c0 + 1, idx1, tf1, rows1, sem1, osem1)
        compute(c0, idx0, tf0, rows0, sem0, osem0)

        @pl.when(c0 + 2 < NCHUNK)
        def _():
            issue(c0 + 2, idx0, tf0, rows0, sem0, osem0)

        compute(c0 + 1, idx1, tf1, rows1, sem1, osem1)
        return carry

    lax.fori_loop(0, NCHUNK // 2, body, 0)
    pltpu.make_async_copy(rows0, out_hbm.at[pl.ds(tile_base, T)], osem0).wait()
    pltpu.make_async_copy(rows1, out_hbm.at[pl.ds(tile_base, T)], osem1).wait()


def kernel(x, token_type_ids, word_table, pos_table, type_table, gamma, beta):
    ids = x.reshape(-1).astype(jnp.int32)
    tf = token_type_ids.reshape(-1).astype(jnp.float32)
    p0 = pos_table + type_table[0]
    dvec = type_table[1] - type_table[0]
    out = _emb_kernel(ids, tf, word_table, p0, dvec, gamma, beta)
    return out.reshape(B, S, D)
